# baseline (speedup 1.0000x reference)
import functools

import jax
import jax.numpy as jnp
from jax.experimental import pallas as pl

_K = 20
_R = 128
_ITERS = 12
_G = 32


def _body(ql_ref, c_ref, feats_ref, sf_ref, ofg_ref, obg_ref):
    feats = feats_ref[...]
    sf = sf_ref[...]
    n_rows = feats.shape[0]
    sim = jax.lax.dot_general(
        feats, sf, (((1,), (0,)), ((), ())),
        preferred_element_type=jnp.float32)

    qn2 = jnp.sum(feats * feats, axis=1, keepdims=True)
    qn_inv = 1.0 / jnp.maximum(jnp.sqrt(qn2), 1e-12)
    sim = sim * qn_inv

    c_row = c_ref[0:1, :]
    mask = jnp.all(ql_ref[...] == c_row, axis=1, keepdims=True)
    mf = mask.astype(jnp.float32)
    mb = 1.0 - mf

    cnt_f = jnp.sum(mf)
    cnt_b = jnp.float32(n_rows) - cnt_f
    kf = jnp.float32(_K)
    n_f = jnp.minimum(kf, cnt_f)
    n_b = jnp.minimum(kf, cnt_b)

    sn2 = jnp.sum(sf * sf, axis=0, keepdims=True)
    sn = jnp.sqrt(sn2)
    neg = jnp.float32(-jnp.inf)

    big = 4.0 * sn
    z = jnp.where(mask, sim, sim - big)

    zg = z.reshape(_G, n_rows // _G, _R)
    gmf = jnp.max(jnp.where(mask.reshape(_G, n_rows // _G, 1), zg, neg),
                  axis=1)
    gmb = jnp.max(jnp.where(mask.reshape(_G, n_rows // _G, 1), neg, zg),
                  axis=1)

    hi_f = jnp.max(gmf, axis=0, keepdims=True)
    hi_b = jnp.max(gmb, axis=0, keepdims=True)

    def drop_max(_, gm):
        m = jnp.max(gm, axis=0, keepdims=True)
        return jnp.where(gm == m, neg, gm)

    gmf = jax.lax.fori_loop(0, _K - 1, drop_max, gmf)
    gmb = jax.lax.fori_loop(0, _K - 1, drop_max, gmb)
    lo_f = jnp.maximum(jnp.max(gmf, axis=0, keepdims=True), -sn)
    lo_b = jnp.maximum(jnp.max(gmb, axis=0, keepdims=True), -5.0 * sn)

    nz_f = n_f
    nz_b = n_b + cnt_f

    def it(_, st):
        lo_f, hi_f, lo_b, hi_b = st
        mid_f = 0.5 * (lo_f + hi_f)
        mid_b = 0.5 * (lo_b + hi_b)
        cf = jnp.sum((z >= mid_f).astype(jnp.float32), axis=0, keepdims=True)
        cb = jnp.sum((z >= mid_b).astype(jnp.float32), axis=0, keepdims=True)
        pf = cf >= nz_f
        pb = cb >= nz_b
        lo_f = jnp.where(pf, mid_f, lo_f)
        hi_f = jnp.where(pf, hi_f, mid_f)
        lo_b = jnp.where(pb, mid_b, lo_b)
        hi_b = jnp.where(pb, hi_b, mid_b)
        return lo_f, hi_f, lo_b, hi_b

    lo_f, hi_f, lo_b, hi_b = jax.lax.fori_loop(
        0, _ITERS, it, (lo_f, hi_f, lo_b, hi_b))
    lo_b = lo_b + big

    gtf = (sim > lo_f).astype(jnp.float32) * mf
    gtb = (sim > lo_b).astype(jnp.float32) * mb
    s_f = jnp.sum(gtf * sim, axis=0, keepdims=True)
    s_b = jnp.sum(gtb * sim, axis=0, keepdims=True)
    cgf = jnp.sum(gtf, axis=0, keepdims=True)
    cgb = jnp.sum(gtb, axis=0, keepdims=True)

    t_f = jnp.where(lo_f > jnp.float32(-1e38), lo_f, 0.0)
    t_b = jnp.where(lo_b > jnp.float32(-1e38), lo_b, 0.0)
    res_f = jnp.where(n_f > 0,
                      (s_f + (n_f - cgf) * t_f) / jnp.maximum(n_f, 1.0), 0.0)
    res_b = jnp.where(n_b > 0,
                      (s_b + (n_b - cgb) * t_b) / jnp.maximum(n_b, 1.0), 0.0)

    sn_inv = 1.0 / jnp.maximum(sn, 1e-12)
    ofg_ref[...] = (res_f * sn_inv).reshape(1, 1, _R)
    obg_ref[...] = (res_b * sn_inv).reshape(1, 1, _R)


@functools.partial(jax.jit, static_argnums=())
def kernel(query_label, color, q_feat, s_feat):
    Hq, Wq = int(q_feat.shape[2]), int(q_feat.shape[3])
    C = int(q_feat.shape[1])
    N = Hq * Wq
    Hs, Ws = int(s_feat.shape[2]), int(s_feat.shape[3])
    M = Hs * Ws

    feats = q_feat.reshape(C, N).T
    sfm = s_feat.reshape(C, M)

    ql = query_label.reshape(N, 3)
    ql_pad = jnp.pad(ql, ((0, 0), (0, 5)))
    c_pad = jnp.pad(color.reshape(1, 3), ((0, 0), (0, 5)))
    c8 = jnp.broadcast_to(c_pad, (8, 8))

    nblk = M // _R
    out_shape = jax.ShapeDtypeStruct((nblk, 1, _R), jnp.float32)
    ofg, obg = pl.pallas_call(
        _body,
        grid=(nblk,),
        in_specs=[
            pl.BlockSpec((N, 8), lambda i: (0, 0)),
            pl.BlockSpec((8, 8), lambda i: (0, 0)),
            pl.BlockSpec((N, C), lambda i: (0, 0)),
            pl.BlockSpec((C, _R), lambda i: (0, i)),
        ],
        out_specs=[
            pl.BlockSpec((1, 1, _R), lambda i: (i, 0, 0)),
            pl.BlockSpec((1, 1, _R), lambda i: (i, 0, 0)),
        ],
        out_shape=[out_shape, out_shape],
    )(ql_pad, c8, feats, sfm)

    return (ofg.reshape(Hs, Ws), obg.reshape(Hs, Ws))


# speedup vs baseline: 1.4743x; 1.4743x over previous
import functools

import jax
import jax.numpy as jnp
from jax.experimental import pallas as pl

_K = 20
_R = 256
_ITERS = 12
_G = 32


def _body(ql_ref, c_ref, feats_ref, sf_ref, ofg_ref, obg_ref):
    feats = feats_ref[...]
    sf = sf_ref[...]
    n_rows = feats.shape[0]
    sim = jax.lax.dot_general(
        feats, sf, (((1,), (0,)), ((), ())),
        preferred_element_type=jnp.float32)

    qn2 = jnp.sum(feats * feats, axis=1, keepdims=True)
    qn_inv = 1.0 / jnp.maximum(jnp.sqrt(qn2), 1e-12)
    sim = sim * qn_inv

    c_row = c_ref[0:1, :]
    mask = jnp.all(ql_ref[...] == c_row, axis=1, keepdims=True)
    mf = mask.astype(jnp.float32)
    mb = 1.0 - mf

    cnt_f = jnp.sum(mf)
    cnt_b = jnp.float32(n_rows) - cnt_f
    kf = jnp.float32(_K)
    n_f = jnp.minimum(kf, cnt_f)
    n_b = jnp.minimum(kf, cnt_b)

    sn2 = jnp.sum(sf * sf, axis=0, keepdims=True)
    sn = jnp.sqrt(sn2)
    neg = jnp.float32(-jnp.inf)

    big = 4.0 * sn
    z = jnp.where(mask, sim, sim - big)

    zg = z.reshape(_G, n_rows // _G, _R)
    gmf = jnp.max(jnp.where(mask.reshape(_G, n_rows // _G, 1), zg, neg),
                  axis=1)
    gmb = jnp.max(jnp.where(mask.reshape(_G, n_rows // _G, 1), neg, zg),
                  axis=1)

    hi_f = jnp.max(gmf, axis=0, keepdims=True)
    hi_b = jnp.max(gmb, axis=0, keepdims=True)

    lo_f = jnp.maximum(jnp.min(gmf, axis=0, keepdims=True), -sn)
    lo_b = jnp.maximum(jnp.min(gmb, axis=0, keepdims=True), -5.0 * sn)

    nz_f = n_f
    nz_b = n_b + cnt_f

    def it(_, st):
        lo_f, hi_f, lo_b, hi_b = st
        mid_f = 0.5 * (lo_f + hi_f)
        mid_b = 0.5 * (lo_b + hi_b)
        cf = jnp.sum((z >= mid_f).astype(jnp.float32), axis=0, keepdims=True)
        cb = jnp.sum((z >= mid_b).astype(jnp.float32), axis=0, keepdims=True)
        pf = cf >= nz_f
        pb = cb >= nz_b
        lo_f = jnp.where(pf, mid_f, lo_f)
        hi_f = jnp.where(pf, hi_f, mid_f)
        lo_b = jnp.where(pb, mid_b, lo_b)
        hi_b = jnp.where(pb, hi_b, mid_b)
        return lo_f, hi_f, lo_b, hi_b

    lo_f, hi_f, lo_b, hi_b = jax.lax.fori_loop(
        0, _ITERS, it, (lo_f, hi_f, lo_b, hi_b))
    lo_b = lo_b + big

    gtf = (sim > lo_f).astype(jnp.float32) * mf
    gtb = (sim > lo_b).astype(jnp.float32) * mb
    s_f = jnp.sum(gtf * sim, axis=0, keepdims=True)
    s_b = jnp.sum(gtb * sim, axis=0, keepdims=True)
    cgf = jnp.sum(gtf, axis=0, keepdims=True)
    cgb = jnp.sum(gtb, axis=0, keepdims=True)

    t_f = jnp.where(lo_f > jnp.float32(-1e38), lo_f, 0.0)
    t_b = jnp.where(lo_b > jnp.float32(-1e38), lo_b, 0.0)
    res_f = jnp.where(n_f > 0,
                      (s_f + (n_f - cgf) * t_f) / jnp.maximum(n_f, 1.0), 0.0)
    res_b = jnp.where(n_b > 0,
                      (s_b + (n_b - cgb) * t_b) / jnp.maximum(n_b, 1.0), 0.0)

    sn_inv = 1.0 / jnp.maximum(sn, 1e-12)
    ofg_ref[...] = (res_f * sn_inv).reshape(1, 1, _R)
    obg_ref[...] = (res_b * sn_inv).reshape(1, 1, _R)


@functools.partial(jax.jit, static_argnums=())
def kernel(query_label, color, q_feat, s_feat):
    Hq, Wq = int(q_feat.shape[2]), int(q_feat.shape[3])
    C = int(q_feat.shape[1])
    N = Hq * Wq
    Hs, Ws = int(s_feat.shape[2]), int(s_feat.shape[3])
    M = Hs * Ws

    feats = q_feat.reshape(C, N).T
    sfm = s_feat.reshape(C, M)

    ql = query_label.reshape(N, 3)
    ql_pad = jnp.pad(ql, ((0, 0), (0, 5)))
    c_pad = jnp.pad(color.reshape(1, 3), ((0, 0), (0, 5)))
    c8 = jnp.broadcast_to(c_pad, (8, 8))

    nblk = M // _R
    out_shape = jax.ShapeDtypeStruct((nblk, 1, _R), jnp.float32)
    ofg, obg = pl.pallas_call(
        _body,
        grid=(nblk,),
        in_specs=[
            pl.BlockSpec((N, 8), lambda i: (0, 0)),
            pl.BlockSpec((8, 8), lambda i: (0, 0)),
            pl.BlockSpec((N, C), lambda i: (0, 0)),
            pl.BlockSpec((C, _R), lambda i: (0, i)),
        ],
        out_specs=[
            pl.BlockSpec((1, 1, _R), lambda i: (i, 0, 0)),
            pl.BlockSpec((1, 1, _R), lambda i: (i, 0, 0)),
        ],
        out_shape=[out_shape, out_shape],
    )(ql_pad, c8, feats, sfm)

    return (ofg.reshape(Hs, Ws), obg.reshape(Hs, Ws))


# speedup vs baseline: 1.8740x; 1.2711x over previous
import functools

import jax
import jax.numpy as jnp
from jax.experimental import pallas as pl

_K = 20
_R = 256
_ITERS = 8
_G = 32


def _body(ql_ref, c_ref, feats_ref, sf_ref, ofg_ref, obg_ref):
    feats = feats_ref[...]
    sf = sf_ref[...]
    n_rows = feats.shape[0]
    sim = jax.lax.dot_general(
        feats, sf, (((1,), (0,)), ((), ())),
        preferred_element_type=jnp.float32)

    qn2 = jnp.sum(feats * feats, axis=1, keepdims=True)
    qn_inv = 1.0 / jnp.maximum(jnp.sqrt(qn2), 1e-12)
    sim = sim * qn_inv

    c_row = c_ref[0:1, :]
    mask = jnp.all(ql_ref[...] == c_row, axis=1, keepdims=True)
    mf = mask.astype(jnp.float32)
    mb = 1.0 - mf

    cnt_f = jnp.sum(mf)
    cnt_b = jnp.float32(n_rows) - cnt_f
    kf = jnp.float32(_K)
    n_f = jnp.minimum(kf, cnt_f)
    n_b = jnp.minimum(kf, cnt_b)

    sn2 = jnp.sum(sf * sf, axis=0, keepdims=True)
    sn = jnp.sqrt(sn2)
    neg = jnp.float32(-jnp.inf)

    big = 4.0 * sn
    z = jnp.where(mask, sim, sim - big)

    zg = z.reshape(_G, n_rows // _G, _R)
    gmf = jnp.max(jnp.where(mask.reshape(_G, n_rows // _G, 1), zg, neg),
                  axis=1)
    gmb = jnp.max(jnp.where(mask.reshape(_G, n_rows // _G, 1), neg, zg),
                  axis=1)

    hi_f = jnp.max(gmf, axis=0, keepdims=True)
    hi_b = jnp.max(gmb, axis=0, keepdims=True)

    lo_f = jnp.maximum(jnp.min(gmf, axis=0, keepdims=True), -sn)
    lo_b = jnp.maximum(jnp.min(gmb, axis=0, keepdims=True), -5.0 * sn)

    nz_f = n_f
    nz_b = n_b + cnt_f

    def it(_, st):
        lo_f, hi_f, lo_b, hi_b = st
        mid_f = 0.5 * (lo_f + hi_f)
        mid_b = 0.5 * (lo_b + hi_b)
        cf = jnp.sum((z >= mid_f).astype(jnp.float32), axis=0, keepdims=True)
        cb = jnp.sum((z >= mid_b).astype(jnp.float32), axis=0, keepdims=True)
        pf = cf >= nz_f
        pb = cb >= nz_b
        lo_f = jnp.where(pf, mid_f, lo_f)
        hi_f = jnp.where(pf, hi_f, mid_f)
        lo_b = jnp.where(pb, mid_b, lo_b)
        hi_b = jnp.where(pb, hi_b, mid_b)
        return lo_f, hi_f, lo_b, hi_b

    lo_f, hi_f, lo_b, hi_b = jax.lax.fori_loop(
        0, _ITERS, it, (lo_f, hi_f, lo_b, hi_b))
    lo_b = lo_b + big

    gtf = (sim > lo_f).astype(jnp.float32) * mf
    gtb = (sim > lo_b).astype(jnp.float32) * mb
    s_f = jnp.sum(gtf * sim, axis=0, keepdims=True)
    s_b = jnp.sum(gtb * sim, axis=0, keepdims=True)
    cgf = jnp.sum(gtf, axis=0, keepdims=True)
    cgb = jnp.sum(gtb, axis=0, keepdims=True)

    t_f = jnp.where(lo_f > jnp.float32(-1e38), lo_f, 0.0)
    t_b = jnp.where(lo_b > jnp.float32(-1e38), lo_b, 0.0)
    res_f = jnp.where(n_f > 0,
                      (s_f + (n_f - cgf) * t_f) / jnp.maximum(n_f, 1.0), 0.0)
    res_b = jnp.where(n_b > 0,
                      (s_b + (n_b - cgb) * t_b) / jnp.maximum(n_b, 1.0), 0.0)

    sn_inv = 1.0 / jnp.maximum(sn, 1e-12)
    ofg_ref[...] = (res_f * sn_inv).reshape(1, 1, _R)
    obg_ref[...] = (res_b * sn_inv).reshape(1, 1, _R)


@functools.partial(jax.jit, static_argnums=())
def kernel(query_label, color, q_feat, s_feat):
    Hq, Wq = int(q_feat.shape[2]), int(q_feat.shape[3])
    C = int(q_feat.shape[1])
    N = Hq * Wq
    Hs, Ws = int(s_feat.shape[2]), int(s_feat.shape[3])
    M = Hs * Ws

    feats = q_feat.reshape(C, N).T
    sfm = s_feat.reshape(C, M)

    ql = query_label.reshape(N, 3)
    ql_pad = jnp.pad(ql, ((0, 0), (0, 5)))
    c_pad = jnp.pad(color.reshape(1, 3), ((0, 0), (0, 5)))
    c8 = jnp.broadcast_to(c_pad, (8, 8))

    nblk = M // _R
    out_shape = jax.ShapeDtypeStruct((nblk, 1, _R), jnp.float32)
    ofg, obg = pl.pallas_call(
        _body,
        grid=(nblk,),
        in_specs=[
            pl.BlockSpec((N, 8), lambda i: (0, 0)),
            pl.BlockSpec((8, 8), lambda i: (0, 0)),
            pl.BlockSpec((N, C), lambda i: (0, 0)),
            pl.BlockSpec((C, _R), lambda i: (0, i)),
        ],
        out_specs=[
            pl.BlockSpec((1, 1, _R), lambda i: (i, 0, 0)),
            pl.BlockSpec((1, 1, _R), lambda i: (i, 0, 0)),
        ],
        out_shape=[out_shape, out_shape],
    )(ql_pad, c8, feats, sfm)

    return (ofg.reshape(Hs, Ws), obg.reshape(Hs, Ws))


# speedup vs baseline: 2.0281x; 1.0822x over previous
import functools

import jax
import jax.numpy as jnp
from jax.experimental import pallas as pl
from jax.experimental.pallas import tpu as pltpu

_K = 20
_R = 256
_ITERS = 8
_G = 32


def _body(ql_ref, c_ref, feats_ref, sf_ref, ofg_ref, obg_ref, fn_ref):
    sf = sf_ref[...]
    n_rows = fn_ref.shape[0]

    @pl.when(pl.program_id(0) == 0)
    def _():
        feats = feats_ref[...]
        qn2 = jnp.sum(feats * feats, axis=1, keepdims=True)
        qn_inv = 1.0 / jnp.maximum(jnp.sqrt(qn2), 1e-12)
        fn_ref[...] = feats * qn_inv

    sim = jax.lax.dot_general(
        fn_ref[...], sf, (((1,), (0,)), ((), ())),
        preferred_element_type=jnp.float32)

    c_row = c_ref[0:1, :]
    mask = jnp.all(ql_ref[...] == c_row, axis=1, keepdims=True)
    mf = mask.astype(jnp.float32)
    mb = 1.0 - mf

    cnt_f = jnp.sum(mf)
    cnt_b = jnp.float32(n_rows) - cnt_f
    kf = jnp.float32(_K)
    n_f = jnp.minimum(kf, cnt_f)
    n_b = jnp.minimum(kf, cnt_b)

    sn2 = jnp.sum(sf * sf, axis=0, keepdims=True)
    sn = jnp.sqrt(sn2)
    neg = jnp.float32(-jnp.inf)

    big = 4.0 * sn
    z = jnp.where(mask, sim, sim - big)

    zg = z.reshape(_G, n_rows // _G, _R)
    gmf = jnp.max(jnp.where(mask.reshape(_G, n_rows // _G, 1), zg, neg),
                  axis=1)
    gmb = jnp.max(jnp.where(mask.reshape(_G, n_rows // _G, 1), neg, zg),
                  axis=1)

    hi_f = jnp.max(gmf, axis=0, keepdims=True)
    hi_b = jnp.max(gmb, axis=0, keepdims=True)

    lo_f = jnp.maximum(jnp.min(gmf, axis=0, keepdims=True), -sn)
    lo_b = jnp.maximum(jnp.min(gmb, axis=0, keepdims=True), -5.0 * sn)

    nz_f = n_f
    nz_b = n_b + cnt_f

    def it(_, st):
        lo_f, hi_f, lo_b, hi_b = st
        mid_f = 0.5 * (lo_f + hi_f)
        mid_b = 0.5 * (lo_b + hi_b)
        cf = jnp.sum((z >= mid_f).astype(jnp.float32), axis=0, keepdims=True)
        cb = jnp.sum((z >= mid_b).astype(jnp.float32), axis=0, keepdims=True)
        pf = cf >= nz_f
        pb = cb >= nz_b
        lo_f = jnp.where(pf, mid_f, lo_f)
        hi_f = jnp.where(pf, hi_f, mid_f)
        lo_b = jnp.where(pb, mid_b, lo_b)
        hi_b = jnp.where(pb, hi_b, mid_b)
        return lo_f, hi_f, lo_b, hi_b

    lo_f, hi_f, lo_b, hi_b = jax.lax.fori_loop(
        0, _ITERS, it, (lo_f, hi_f, lo_b, hi_b))
    lo_b = lo_b + big

    gtf = (sim > lo_f).astype(jnp.float32) * mf
    gtb = (sim > lo_b).astype(jnp.float32) * mb
    s_f = jnp.sum(gtf * sim, axis=0, keepdims=True)
    s_b = jnp.sum(gtb * sim, axis=0, keepdims=True)
    cgf = jnp.sum(gtf, axis=0, keepdims=True)
    cgb = jnp.sum(gtb, axis=0, keepdims=True)

    t_f = jnp.where(lo_f > jnp.float32(-1e38), lo_f, 0.0)
    t_b = jnp.where(lo_b > jnp.float32(-1e38), lo_b, 0.0)
    res_f = jnp.where(n_f > 0,
                      (s_f + (n_f - cgf) * t_f) / jnp.maximum(n_f, 1.0), 0.0)
    res_b = jnp.where(n_b > 0,
                      (s_b + (n_b - cgb) * t_b) / jnp.maximum(n_b, 1.0), 0.0)

    sn_inv = 1.0 / jnp.maximum(sn, 1e-12)
    ofg_ref[...] = (res_f * sn_inv).reshape(1, 1, _R)
    obg_ref[...] = (res_b * sn_inv).reshape(1, 1, _R)


@functools.partial(jax.jit, static_argnums=())
def kernel(query_label, color, q_feat, s_feat):
    Hq, Wq = int(q_feat.shape[2]), int(q_feat.shape[3])
    C = int(q_feat.shape[1])
    N = Hq * Wq
    Hs, Ws = int(s_feat.shape[2]), int(s_feat.shape[3])
    M = Hs * Ws

    feats = q_feat.reshape(C, N).T
    sfm = s_feat.reshape(C, M)

    ql = query_label.reshape(N, 3)
    ql_pad = jnp.pad(ql, ((0, 0), (0, 5)))
    c_pad = jnp.pad(color.reshape(1, 3), ((0, 0), (0, 5)))
    c8 = jnp.broadcast_to(c_pad, (8, 8))

    nblk = M // _R
    out_shape = jax.ShapeDtypeStruct((nblk, 1, _R), jnp.float32)
    ofg, obg = pl.pallas_call(
        _body,
        grid=(nblk,),
        in_specs=[
            pl.BlockSpec((N, 8), lambda i: (0, 0)),
            pl.BlockSpec((8, 8), lambda i: (0, 0)),
            pl.BlockSpec((N, C), lambda i: (0, 0)),
            pl.BlockSpec((C, _R), lambda i: (0, i)),
        ],
        out_specs=[
            pl.BlockSpec((1, 1, _R), lambda i: (i, 0, 0)),
            pl.BlockSpec((1, 1, _R), lambda i: (i, 0, 0)),
        ],
        out_shape=[out_shape, out_shape],
        scratch_shapes=[pltpu.VMEM((N, C), jnp.float32)],
    )(ql_pad, c8, feats, sfm)

    return (ofg.reshape(Hs, Ws), obg.reshape(Hs, Ws))
